# baseline (speedup 1.0000x reference)
import functools

import jax
import jax.numpy as jnp
from jax import lax
from jax.experimental import pallas as pl
from jax.experimental.pallas import tpu as pltpu
from jax.experimental.pallas import tpu_sc as plsc

B, H, W = 4, 64, 64
R = 3.0
THRESH = 1.05
ROWS = 8
PIX = ROWS * W
NCHUNK = PIX * 4 // 128


def _sc_body(coords_hbm, noise_hbm, corr_hbm, out_hbm, halo_hbm,
             y_ext, x_ext, s_own, cy_b, cx_b, nz, idxb, valb, sem):
    c = lax.axis_index("c")
    s = lax.axis_index("s")
    batch = 2 * c + s // 8
    blk = s % 8
    r0 = blk * ROWS

    lanes = lax.iota(jnp.int32, 16)

    pltpu.sync_copy(coords_hbm.at[batch, 0, pl.ds(r0, ROWS)],
                    y_ext.at[pl.ds(1, ROWS)])
    pltpu.sync_copy(coords_hbm.at[batch, 1, pl.ds(r0, ROWS)],
                    x_ext.at[pl.ds(1, ROWS)])
    for m in range(3):
        for pln in range(2):
            pltpu.sync_copy(noise_hbm.at[m, batch, pln, pl.ds(r0, ROWS)],
                            nz.at[m, pln])

    wid = c * 16 + s

    def exchange():
        pltpu.sync_copy(y_ext.at[1], halo_hbm.at[wid, 0])
        pltpu.sync_copy(x_ext.at[1], halo_hbm.at[wid, 1])
        pltpu.sync_copy(y_ext.at[ROWS], halo_hbm.at[wid, 2])
        pltpu.sync_copy(x_ext.at[ROWS], halo_hbm.at[wid, 3])
        plsc.subcore_barrier()
        sbase = c * 16 + (s // 8) * 8
        s_top = sbase + ((blk + 7) % 8)
        s_bot = sbase + ((blk + 1) % 8)
        pltpu.sync_copy(halo_hbm.at[s_top, 2], y_ext.at[0])
        pltpu.sync_copy(halo_hbm.at[s_top, 3], x_ext.at[0])
        pltpu.sync_copy(halo_hbm.at[s_bot, 0], y_ext.at[ROWS + 1])
        pltpu.sync_copy(halo_hbm.at[s_bot, 1], x_ext.at[ROWS + 1])
        plsc.subcore_barrier()

    def tail(v, r, b_, cy, cx):
        k = v >> 2
        cb = (v & 3) * 16
        cy_b[k, pl.ds(cb, 16)] = cy
        cx_b[k, pl.ds(cb, 16)] = cx
        col = cb + lanes
        n = batch * 4096 + (r0 + k) * 64 + col
        base = n << 12
        y0 = cy.astype(jnp.int32)
        x0 = cx.astype(jnp.int32)
        y1 = jnp.minimum(y0 + 1, H - 1)
        x1 = jnp.minimum(x0 + 1, W - 1)
        rw0 = base + (y0 << 6)
        rw1 = base + (y1 << 6)
        off = b_ * 64
        idxb[r, pl.ds(off, 16)] = rw0 + x0
        idxb[r, pl.ds(off + 16, 16)] = rw0 + x1
        idxb[r, pl.ds(off + 32, 16)] = rw1 + x0
        idxb[r, pl.ds(off + 48, 16)] = rw1 + x1

    def drain_all():
        def dr(r, _):
            pltpu.make_async_copy(corr_hbm.at[idxb.at[r]], valb.at[r],
                                  sem).wait()
            return 0
        lax.fori_loop(0, NCHUNK, dr, 0)

    def drain_update(r, mode, m=0):
        for b_ in range(2):
            v = 2 * r + b_
            k = v >> 2
            cb = (v & 3) * 16
            cy = cy_b[k, pl.ds(cb, 16)]
            cx = cx_b[k, pl.ds(cb, 16)]
            wy = cy - cy.astype(jnp.int32).astype(jnp.float32)
            wx = cx - cx.astype(jnp.int32).astype(jnp.float32)
            off = b_ * 64
            v00 = valb[r, pl.ds(off, 16)]
            v01 = valb[r, pl.ds(off + 16, 16)]
            v10 = valb[r, pl.ds(off + 32, 16)]
            v11 = valb[r, pl.ds(off + 48, 16)]
            sc = (v00 * (1.0 - wy) * (1.0 - wx) + v01 * (1.0 - wy) * wx
                  + v10 * wy * (1.0 - wx) + v11 * wy * wx)
            if mode == "init":
                s_own[k, pl.ds(cb, 16)] = sc
            else:
                sold = s_own[k, pl.ds(cb, 16)]
                if mode == "prop":
                    upd = sc > sold
                else:
                    upd = sc > jnp.float32(THRESH) * sold
                yold = y_ext[1 + k, pl.ds(cb, 16)]
                xold = x_ext[1 + k, pl.ds(cb, 16)]
                y_ext[1 + k, pl.ds(cb, 16)] = jnp.where(upd, cy, yold)
                x_ext[1 + k, pl.ds(cb, 16)] = jnp.where(upd, cx, xold)
                s_own[k, pl.ds(cb, 16)] = jnp.where(upd, sc, sold)

    def eval_init():
        def fire(r, _):
            for b_ in range(2):
                v = 2 * r + b_
                k = v >> 2
                cb = (v & 3) * 16
                cy = y_ext[1 + k, pl.ds(cb, 16)]
                cx = x_ext[1 + k, pl.ds(cb, 16)]
                tail(v, r, b_, cy, cx)
            pltpu.async_copy(corr_hbm.at[idxb.at[r]], valb.at[r], sem)
            return 0
        lax.fori_loop(0, NCHUNK, fire, 0)

        drain_all()

        def upd(r, _):
            drain_update(r, "init")
            return 0
        lax.fori_loop(0, NCHUNK, upd, 0)

    def eval_prop(dy, dx):
        exchange()

        def fire(r, _):
            for b_ in range(2):
                v = 2 * r + b_
                k = v >> 2
                cb = (v & 3) * 16
                col = cb + lanes
                srow = jnp.broadcast_to(k + (1 - dy), (16,)).astype(jnp.int32)
                scol = (col - dx) & 63
                gy = plsc.load_gather(y_ext, [srow, scol])
                gx = plsc.load_gather(x_ext, [srow, scol])
                cy = jnp.minimum(jnp.maximum(gy + jnp.float32(dy), 0.0),
                                 jnp.float32(H - 1))
                cx = jnp.minimum(jnp.maximum(gx + jnp.float32(dx), 0.0),
                                 jnp.float32(W - 1))
                tail(v, r, b_, cy, cx)
            pltpu.async_copy(corr_hbm.at[idxb.at[r]], valb.at[r], sem)
            return 0
        lax.fori_loop(0, NCHUNK, fire, 0)

        drain_all()

        def upd(r, _):
            drain_update(r, "prop")
            return 0
        lax.fori_loop(0, NCHUNK, upd, 0)

    def eval_rand(m):
        def fire(r, _):
            for b_ in range(2):
                v = 2 * r + b_
                k = v >> 2
                cb = (v & 3) * 16
                ny = y_ext[1 + k, pl.ds(cb, 16)] + nz[m, 0, k, pl.ds(cb, 16)]
                nx = x_ext[1 + k, pl.ds(cb, 16)] + nz[m, 1, k, pl.ds(cb, 16)]
                ny = jnp.where(ny < 0.0, 0.0, ny)
                nx = jnp.where(nx < 0.0, 0.0, nx)
                mh = ny >= jnp.float32(H)
                ny = jnp.where(mh, jnp.float32(H - 1), ny)
                nx = jnp.where(mh, jnp.float32(H - 1), nx)
                mw = nx >= jnp.float32(W)
                ny = jnp.where(mw, jnp.float32(W - 1), ny)
                nx = jnp.where(mw, jnp.float32(W - 1), nx)
                tail(v, r, b_, ny, nx)
            pltpu.async_copy(corr_hbm.at[idxb.at[r]], valb.at[r], sem)
            return 0
        lax.fori_loop(0, NCHUNK, fire, 0)

        drain_all()

        def upd(r, _):
            drain_update(r, "rand")
            return 0
        lax.fori_loop(0, NCHUNK, upd, 0)

    eval_init()
    eval_prop(1, 1)
    eval_rand(0)
    eval_prop(-1, -1)
    eval_rand(1)
    eval_prop(-1, 1)
    eval_rand(2)
    eval_prop(1, -1)

    pltpu.sync_copy(y_ext.at[pl.ds(1, ROWS)],
                    out_hbm.at[batch, 0, pl.ds(r0, ROWS)])
    pltpu.sync_copy(x_ext.at[pl.ds(1, ROWS)],
                    out_hbm.at[batch, 1, pl.ds(r0, ROWS)])


@jax.jit
def _run(raw_coords, noise_t, corr_flat):
    mesh = plsc.VectorSubcoreMesh(core_axis_name="c", subcore_axis_name="s")
    f = pl.kernel(
        _sc_body,
        out_type=(jax.ShapeDtypeStruct((B, 2, H, W), jnp.float32),
                  jax.ShapeDtypeStruct((32, 4, W), jnp.float32)),
        mesh=mesh,
        compiler_params=pltpu.CompilerParams(needs_layout_passes=False),
        scratch_types=[
            pltpu.VMEM((ROWS + 2, W), jnp.float32),
            pltpu.VMEM((ROWS + 2, W), jnp.float32),
            pltpu.VMEM((ROWS, W), jnp.float32),
            pltpu.VMEM((ROWS, W), jnp.float32),
            pltpu.VMEM((ROWS, W), jnp.float32),
            pltpu.VMEM((3, 2, ROWS, W), jnp.float32),
            pltpu.VMEM((NCHUNK, 128), jnp.int32),
            pltpu.VMEM((NCHUNK, 128), jnp.float32),
            pltpu.SemaphoreType.DMA,
        ],
    )
    out, _halo = f(raw_coords, noise_t, corr_flat)
    return out


def kernel(raw_coords, corr_map):
    key = jax.random.key(42)
    ks = jax.random.split(key, 3)
    noise = jnp.stack(
        [jax.random.normal(k, (B, H, W, 2), jnp.float32) * R for k in ks])
    noise_t = jnp.transpose(noise, (0, 1, 4, 2, 3))
    corr_flat = corr_map.reshape(-1)
    return _run(raw_coords, noise_t, corr_flat)
